# Initial kernel scaffold; baseline (speedup 1.0000x reference)
import jax
import jax.numpy as jnp
from jax.experimental import pallas as pl


def kernel(x, edge_index_follows, edge_index_likes, W1_follows, b1_follows, W1_likes, b1_likes, W2_follows, b2_follows, W2_likes, b2_likes):
    raise NotImplementedError("write your pallas kernel here")


# speedup vs baseline: 8.9138x; 8.9138x over previous
import functools

import jax
import jax.numpy as jnp
from jax import lax
from jax.experimental import pallas as pl
from jax.experimental.pallas import tpu as pltpu
from jax.experimental.pallas import tpu_sc as plsc

N = 10000
E = 320000
DIN = 128
H = 16
NCLS = 2

NC = 2
NS = 16
LANES = 16

CHUNK = 128
NCHUNKS = E // CHUNK
ROWS_PER_TILE = N // NS
CNT_STEP = 624
CNT_LEN = 640


_BLK = 2000
_GRID = N // _BLK


def _linear1(x, W1, b1):
    def body(x_ref, w_ref, b_ref, of_ref, ol_ref):
        res = jnp.dot(x_ref[...], w_ref[...],
                      preferred_element_type=jnp.float32) + b_ref[...]
        of_ref[...] = res[:, :H]
        ol_ref[...] = res[:, H:]

    return pl.pallas_call(
        body,
        grid=(_GRID,),
        in_specs=[
            pl.BlockSpec((_BLK, DIN), lambda i: (i, 0)),
            pl.BlockSpec((DIN, 2 * H), lambda i: (0, 0)),
            pl.BlockSpec((1, 2 * H), lambda i: (0, 0)),
        ],
        out_specs=[pl.BlockSpec((_BLK, H), lambda i: (i, 0))] * 2,
        out_shape=[jax.ShapeDtypeStruct((N, H), jnp.float32)] * 2,
    )(x, W1, b1)


def _mid(accf, accl, cntf, cntl, W2f, b2f, W2l, b2l):
    def body(af, al, cf, cl, wf, bf, wl, bl, h1a_ref, of_ref, ol_ref):
        h1 = (af[...] / jnp.maximum(cf[...], 1.0)
              + al[...] / jnp.maximum(cl[...], 1.0))
        h1a = jnp.where(h1 >= 0.0, h1, 0.01 * h1)
        h1a_ref[...] = h1a
        of_ref[...] = jnp.dot(h1a, wf[...],
                              preferred_element_type=jnp.float32) + bf[...]
        ol_ref[...] = jnp.dot(h1a, wl[...],
                              preferred_element_type=jnp.float32) + bl[...]

    row = lambda i: (i, 0)
    full = lambda i: (0, 0)
    return pl.pallas_call(
        body,
        grid=(_GRID,),
        in_specs=[
            pl.BlockSpec((_BLK, H), row),
            pl.BlockSpec((_BLK, H), row),
            pl.BlockSpec((_BLK, 1), row),
            pl.BlockSpec((_BLK, 1), row),
            pl.BlockSpec((H, H), full),
            pl.BlockSpec((1, H), full),
            pl.BlockSpec((H, H), full),
            pl.BlockSpec((1, H), full),
        ],
        out_specs=[pl.BlockSpec((_BLK, H), row)] * 3,
        out_shape=[jax.ShapeDtypeStruct((N, H), jnp.float32)] * 3,
    )(accf, accl, cntf, cntl, W2f, b2f, W2l, b2l)


def _final(accf, accl, cntf, cntl):
    def body(af, al, cf, cl, out_ref):
        out_ref[...] = (af[...][:, :NCLS] / jnp.maximum(cf[...], 1.0)
                        + al[...][:, :NCLS] / jnp.maximum(cl[...], 1.0))

    row = lambda i: (i, 0)
    return pl.pallas_call(
        body,
        grid=(_GRID,),
        in_specs=[
            pl.BlockSpec((_BLK, H), row),
            pl.BlockSpec((_BLK, H), row),
            pl.BlockSpec((_BLK, 1), row),
            pl.BlockSpec((_BLK, 1), row),
        ],
        out_specs=pl.BlockSpec((_BLK, NCLS), row),
        out_shape=jax.ShapeDtypeStruct((N, NCLS), jnp.float32),
    )(accf, accl, cntf, cntl)


@functools.lru_cache(maxsize=None)
def _make_agg(with_counts):
    mesh = plsc.VectorSubcoreMesh(core_axis_name="c", subcore_axis_name="s")

    if with_counts:
        out_type = [jax.ShapeDtypeStruct((2, N, H), jnp.float32),
                    jax.ShapeDtypeStruct((2, N), jnp.float32)]
    else:
        out_type = jax.ShapeDtypeStruct((2, N, H), jnp.float32)

    scratch = [
        pltpu.VMEM((1, CHUNK), jnp.int32),
        pltpu.VMEM((1, CHUNK), jnp.int32),
        pltpu.VMEM((CHUNK, H), jnp.float32),
        pltpu.VMEM((CHUNK,), jnp.float32),
        pltpu.VMEM((ROWS_PER_TILE, H), jnp.float32),
        pltpu.VMEM((CNT_LEN,), jnp.float32),
        pltpu.VMEM_SHARED((N, H), jnp.float32),
        pltpu.VMEM_SHARED((N,), jnp.float32),
        pltpu.SemaphoreType.DMA,
    ]

    def body(tblf, tbll, eif, eil, *refs):
        if with_counts:
            acc_out, cnt_out = refs[0], refs[1]
            refs = refs[2:]
        else:
            acc_out, cnt_out = refs[0], None
            refs = refs[1:]
        srcb, dstb, rows, ones, zbuf, zbuf1, acc_sp, cnt_sp, sem = refs

        c = lax.axis_index("c")
        s = lax.axis_index("s")

        zero16 = jnp.zeros((LANES,), jnp.float32)

        def zrow(i, carry):
            zbuf[i, :] = zero16
            return carry
        lax.fori_loop(0, ROWS_PER_TILE, zrow, 0)
        pltpu.sync_copy(zbuf, acc_sp.at[pl.ds(s * ROWS_PER_TILE,
                                              ROWS_PER_TILE)])
        if with_counts:
            for i in range(CNT_LEN // LANES):
                zbuf1[pl.ds(i * LANES, LANES)] = zero16
            pltpu.sync_copy(zbuf1, cnt_sp.at[pl.ds(s * CNT_STEP, CNT_LEN)])
            one16 = jnp.ones((LANES,), jnp.float32)
            for i in range(CHUNK // LANES):
                ones[pl.ds(i * LANES, LANES)] = one16

        plsc.subcore_barrier()

        def process(tbl, ei):
            nj = jnp.where(s < NCHUNKS % NS, NCHUNKS // NS + 1, NCHUNKS // NS)

            def chunk(j, carry):
                ci = s + j * NS
                pltpu.sync_copy(ei.at[0, ci], srcb.at[0])
                pltpu.sync_copy(ei.at[1, ci], dstb.at[0])
                pltpu.async_copy(tbl.at[srcb.at[0]], rows, sem).wait()
                pltpu.sync_copy(rows, acc_sp.at[dstb.at[0]], add=True)
                if with_counts:
                    pltpu.sync_copy(ones, cnt_sp.at[dstb.at[0]], add=True)
                return carry
            lax.fori_loop(0, nj, chunk, 0)

        @pl.when(c == 0)
        def _():
            process(tblf, eif)

        @pl.when(c == 1)
        def _():
            process(tbll, eil)

        plsc.subcore_barrier()

        pltpu.sync_copy(acc_sp.at[pl.ds(s * ROWS_PER_TILE, ROWS_PER_TILE)],
                        zbuf)
        pltpu.sync_copy(zbuf,
                        acc_out.at[c, pl.ds(s * ROWS_PER_TILE,
                                            ROWS_PER_TILE)])
        if with_counts:
            pltpu.sync_copy(cnt_sp.at[pl.ds(s * CNT_STEP, CNT_LEN)], zbuf1)
            pltpu.sync_copy(zbuf1, cnt_out.at[c, pl.ds(s * CNT_STEP,
                                                       CNT_LEN)])

    return pl.kernel(body, out_type=out_type, mesh=mesh,
                     scratch_types=scratch,
                     compiler_params=pltpu.CompilerParams(
                         use_tc_tiling_on_sc=False))


def kernel(x, edge_index_follows, edge_index_likes,
           W1_follows, b1_follows, W1_likes, b1_likes,
           W2_follows, b2_follows, W2_likes, b2_likes):
    eif = edge_index_follows.reshape(2, NCHUNKS, CHUNK)
    eil = edge_index_likes.reshape(2, NCHUNKS, CHUNK)

    W1 = jnp.concatenate([W1_follows, W1_likes], axis=1)
    b1 = jnp.concatenate([b1_follows, b1_likes])[None, :]
    tbl1_f, tbl1_l = _linear1(x, W1, b1)

    acc1, cnt = _make_agg(True)(tbl1_f, tbl1_l, eif, eil)
    cntf = cnt[0][:, None]
    cntl = cnt[1][:, None]

    pad = ((0, 0), (0, H - NCLS))
    W2f = jnp.pad(W2_follows, pad)
    W2l = jnp.pad(W2_likes, pad)
    b2f = jnp.pad(b2_follows, (0, H - NCLS))[None, :]
    b2l = jnp.pad(b2_likes, (0, H - NCLS))[None, :]

    h1_act, tbl2_f, tbl2_l = _mid(acc1[0], acc1[1], cntf, cntl,
                                  W2f, b2f, W2l, b2l)

    acc2 = _make_agg(False)(tbl2_f, tbl2_l, eif, eil)

    h2 = _final(acc2[0], acc2[1], cntf, cntl)
    return (h2, h1_act)


# speedup vs baseline: 32.2708x; 3.6203x over previous
import functools

import jax
import jax.numpy as jnp
from jax import lax
from jax.experimental import pallas as pl
from jax.experimental.pallas import tpu as pltpu
from jax.experimental.pallas import tpu_sc as plsc

N = 10000
E = 320000
DIN = 128
H = 16
NCLS = 2

NC = 2
NS = 16
LANES = 16

CHUNK = 128
NCHUNKS = E // CHUNK
PT = NCHUNKS // NS
LEFT = NCHUNKS - PT * NS
K = 12
NG = PT // K
ROWS_PER_TILE = N // NS
CNT_STEP = 624
CNT_LEN = 640


_BLK = 2000
_GRID = N // _BLK


def _linear1(x, W1, b1):
    def body(x_ref, w_ref, b_ref, of_ref, ol_ref):
        res = jnp.dot(x_ref[...], w_ref[...],
                      preferred_element_type=jnp.float32) + b_ref[...]
        of_ref[...] = res[:, :H]
        ol_ref[...] = res[:, H:]

    return pl.pallas_call(
        body,
        grid=(_GRID,),
        in_specs=[
            pl.BlockSpec((_BLK, DIN), lambda i: (i, 0)),
            pl.BlockSpec((DIN, 2 * H), lambda i: (0, 0)),
            pl.BlockSpec((1, 2 * H), lambda i: (0, 0)),
        ],
        out_specs=[pl.BlockSpec((_BLK, H), lambda i: (i, 0))] * 2,
        out_shape=[jax.ShapeDtypeStruct((N, H), jnp.float32)] * 2,
    )(x, W1, b1)


def _mid(accf, accl, cntf, cntl, W2f, b2f, W2l, b2l):
    def body(af, al, cf, cl, wf, bf, wl, bl, h1a_ref, of_ref, ol_ref):
        h1 = (af[...] / jnp.maximum(cf[...], 1.0)
              + al[...] / jnp.maximum(cl[...], 1.0))
        h1a = jnp.where(h1 >= 0.0, h1, 0.01 * h1)
        h1a_ref[...] = h1a
        of_ref[...] = jnp.dot(h1a, wf[...],
                              preferred_element_type=jnp.float32) + bf[...]
        ol_ref[...] = jnp.dot(h1a, wl[...],
                              preferred_element_type=jnp.float32) + bl[...]

    row = lambda i: (i, 0)
    full = lambda i: (0, 0)
    return pl.pallas_call(
        body,
        grid=(_GRID,),
        in_specs=[
            pl.BlockSpec((_BLK, H), row),
            pl.BlockSpec((_BLK, H), row),
            pl.BlockSpec((_BLK, 1), row),
            pl.BlockSpec((_BLK, 1), row),
            pl.BlockSpec((H, H), full),
            pl.BlockSpec((1, H), full),
            pl.BlockSpec((H, H), full),
            pl.BlockSpec((1, H), full),
        ],
        out_specs=[pl.BlockSpec((_BLK, H), row)] * 3,
        out_shape=[jax.ShapeDtypeStruct((N, H), jnp.float32)] * 3,
    )(accf, accl, cntf, cntl, W2f, b2f, W2l, b2l)


def _final(accf, accl, cntf, cntl):
    def body(af, al, cf, cl, out_ref):
        out_ref[...] = (af[...][:, :NCLS] / jnp.maximum(cf[...], 1.0)
                        + al[...][:, :NCLS] / jnp.maximum(cl[...], 1.0))

    row = lambda i: (i, 0)
    return pl.pallas_call(
        body,
        grid=(_GRID,),
        in_specs=[
            pl.BlockSpec((_BLK, H), row),
            pl.BlockSpec((_BLK, H), row),
            pl.BlockSpec((_BLK, 1), row),
            pl.BlockSpec((_BLK, 1), row),
        ],
        out_specs=pl.BlockSpec((_BLK, NCLS), row),
        out_shape=jax.ShapeDtypeStruct((N, NCLS), jnp.float32),
    )(accf, accl, cntf, cntl)


@functools.lru_cache(maxsize=None)
def _make_agg(with_counts):
    mesh = plsc.VectorSubcoreMesh(core_axis_name="c", subcore_axis_name="s")

    if with_counts:
        out_type = [jax.ShapeDtypeStruct((2, N, H), jnp.float32),
                    jax.ShapeDtypeStruct((2, N), jnp.float32)]
    else:
        out_type = jax.ShapeDtypeStruct((2, N, H), jnp.float32)

    scratch = [
        pltpu.VMEM((1, CHUNK), jnp.int32),
        pltpu.VMEM((1, CHUNK), jnp.int32),
        pltpu.VMEM((PT, CHUNK), jnp.int32),
        pltpu.VMEM((PT, CHUNK), jnp.int32),
        pltpu.VMEM((2, K, CHUNK, H), jnp.float32),
        pltpu.VMEM((CHUNK,), jnp.float32),
        pltpu.VMEM((ROWS_PER_TILE, H), jnp.float32),
        pltpu.VMEM((CNT_LEN,), jnp.float32),
        pltpu.VMEM_SHARED((N, H), jnp.float32),
        pltpu.VMEM_SHARED((N,), jnp.float32),
        pltpu.SemaphoreType.DMA((2,)),
        pltpu.SemaphoreType.DMA((2,)),
        pltpu.SemaphoreType.DMA,
    ]

    def body(tblf, tbll, eif, eil, *refs):
        if with_counts:
            acc_out, cnt_out = refs[0], refs[1]
            refs = refs[2:]
        else:
            acc_out, cnt_out = refs[0], None
            refs = refs[1:]
        (srcb, dstb, src_all, dst_all, ring, ones, zbuf, zbuf1,
         acc_sp, cnt_sp, sem_g, sem_s, sem_c) = refs

        c = lax.axis_index("c")
        s = lax.axis_index("s")

        zero16 = jnp.zeros((LANES,), jnp.float32)

        def zrow(i, carry):
            zbuf[i, :] = zero16
            return carry
        lax.fori_loop(0, ROWS_PER_TILE, zrow, 0)
        pltpu.sync_copy(zbuf, acc_sp.at[pl.ds(s * ROWS_PER_TILE,
                                              ROWS_PER_TILE)])
        if with_counts:
            for i in range(CNT_LEN // LANES):
                zbuf1[pl.ds(i * LANES, LANES)] = zero16
            pltpu.sync_copy(zbuf1, cnt_sp.at[pl.ds(s * CNT_STEP, CNT_LEN)])
            one16 = jnp.ones((LANES,), jnp.float32)
            for i in range(CHUNK // LANES):
                ones[pl.ds(i * LANES, LANES)] = one16

        plsc.subcore_barrier()

        def process(tbl, ei):
            pltpu.sync_copy(ei.at[0, pl.ds(s * PT, PT)], src_all)
            pltpu.sync_copy(ei.at[1, pl.ds(s * PT, PT)], dst_all)

            def issue_gathers(g):
                h = g % 2
                for k in range(K):
                    pltpu.async_copy(tbl.at[src_all.at[g * K + k]],
                                     ring.at[h, k], sem_g.at[h])

            def wait_gathers(h):
                for k in range(K):
                    pltpu.make_async_copy(tbl.at[src_all.at[k]],
                                          ring.at[0, k], sem_g.at[h]).wait()

            def issue_scatters(g):
                h = g % 2
                for k in range(K):
                    pltpu.async_copy(ring.at[h, k],
                                     acc_sp.at[dst_all.at[g * K + k]],
                                     sem_s.at[h], add=True)

            def wait_scatters(h):
                for k in range(K):
                    pltpu.make_async_copy(ring.at[0, k],
                                          acc_sp.at[dst_all.at[k]],
                                          sem_s.at[h]).wait()

            def issue_cnts(g):
                for k in range(K):
                    pltpu.async_copy(ones,
                                     cnt_sp.at[dst_all.at[g * K + k]],
                                     sem_c, add=True)

            def wait_cnts():
                for k in range(K):
                    pltpu.make_async_copy(ones, cnt_sp.at[dst_all.at[k]],
                                          sem_c).wait()

            issue_gathers(0)

            def grp(g, carry):
                h = g % 2
                o = (g + 1) % 2
                @pl.when(g >= 1)
                def _():
                    wait_scatters(o)
                @pl.when(g + 1 < NG)
                def _():
                    issue_gathers(g + 1)
                wait_gathers(h)
                issue_scatters(g)
                if with_counts:
                    issue_cnts(g)
                    @pl.when(g >= 1)
                    def _():
                        wait_cnts()
                return carry
            lax.fori_loop(0, NG, grp, 0)

            wait_scatters((NG - 1) % 2)
            if with_counts:
                wait_cnts()

            @pl.when(s < LEFT)
            def _():
                ci = NS * PT + s
                pltpu.sync_copy(ei.at[0, ci], srcb.at[0])
                pltpu.sync_copy(ei.at[1, ci], dstb.at[0])
                pltpu.async_copy(tbl.at[srcb.at[0]], ring.at[0, 0],
                                 sem_g.at[0]).wait()
                pltpu.sync_copy(ring.at[0, 0], acc_sp.at[dstb.at[0]],
                                add=True)
                if with_counts:
                    pltpu.sync_copy(ones, cnt_sp.at[dstb.at[0]], add=True)

        @pl.when(c == 0)
        def _():
            process(tblf, eif)

        @pl.when(c == 1)
        def _():
            process(tbll, eil)

        plsc.subcore_barrier()

        pltpu.sync_copy(acc_sp.at[pl.ds(s * ROWS_PER_TILE, ROWS_PER_TILE)],
                        zbuf)
        pltpu.sync_copy(zbuf,
                        acc_out.at[c, pl.ds(s * ROWS_PER_TILE,
                                            ROWS_PER_TILE)])
        if with_counts:
            pltpu.sync_copy(cnt_sp.at[pl.ds(s * CNT_STEP, CNT_LEN)], zbuf1)
            pltpu.sync_copy(zbuf1, cnt_out.at[c, pl.ds(s * CNT_STEP,
                                                       CNT_LEN)])

    return pl.kernel(body, out_type=out_type, mesh=mesh,
                     scratch_types=scratch,
                     compiler_params=pltpu.CompilerParams(
                         use_tc_tiling_on_sc=False))


def kernel(x, edge_index_follows, edge_index_likes,
           W1_follows, b1_follows, W1_likes, b1_likes,
           W2_follows, b2_follows, W2_likes, b2_likes):
    eif = edge_index_follows.reshape(2, NCHUNKS, CHUNK)
    eil = edge_index_likes.reshape(2, NCHUNKS, CHUNK)

    W1 = jnp.concatenate([W1_follows, W1_likes], axis=1)
    b1 = jnp.concatenate([b1_follows, b1_likes])[None, :]
    tbl1_f, tbl1_l = _linear1(x, W1, b1)

    acc1, cnt = _make_agg(True)(tbl1_f, tbl1_l, eif, eil)
    cntf = cnt[0][:, None]
    cntl = cnt[1][:, None]

    pad = ((0, 0), (0, H - NCLS))
    W2f = jnp.pad(W2_follows, pad)
    W2l = jnp.pad(W2_likes, pad)
    b2f = jnp.pad(b2_follows, (0, H - NCLS))[None, :]
    b2l = jnp.pad(b2_likes, (0, H - NCLS))[None, :]

    h1_act, tbl2_f, tbl2_l = _mid(acc1[0], acc1[1], cntf, cntl,
                                  W2f, b2f, W2l, b2l)

    acc2 = _make_agg(False)(tbl2_f, tbl2_l, eif, eil)

    h2 = _final(acc2[0], acc2[1], cntf, cntl)
    return (h2, h1_act)


# speedup vs baseline: 36.5528x; 1.1327x over previous
import functools

import jax
import jax.numpy as jnp
from jax import lax
from jax.experimental import pallas as pl
from jax.experimental.pallas import tpu as pltpu
from jax.experimental.pallas import tpu_sc as plsc

N = 10000
E = 320000
DIN = 128
H = 16
NCLS = 2

NC = 2
NS = 16
LANES = 16

CHUNK = 128
NCHUNKS = E // CHUNK
PT = NCHUNKS // NS
LEFT = NCHUNKS - PT * NS
K = 12
NG = PT // K
ROWS_PER_TILE = 632


_BLK = 2000
_GRID = N // _BLK


def _linear1(x, W1, b1):
    def body(x_ref, w_ref, b_ref, of_ref, ol_ref):
        res = jnp.dot(x_ref[...], w_ref[...],
                      preferred_element_type=jnp.float32) + b_ref[...]
        of_ref[...] = res[:, :H]
        ol_ref[...] = res[:, H:]

    return pl.pallas_call(
        body,
        grid=(_GRID,),
        in_specs=[
            pl.BlockSpec((_BLK, DIN), lambda i: (i, 0)),
            pl.BlockSpec((DIN, 2 * H), lambda i: (0, 0)),
            pl.BlockSpec((1, 2 * H), lambda i: (0, 0)),
        ],
        out_specs=[pl.BlockSpec((_BLK, H), lambda i: (i, 0))] * 2,
        out_shape=[jax.ShapeDtypeStruct((N, H), jnp.float32)] * 2,
    )(x, W1, b1)


def _mid(acc, W2f, b2f, W2l, b2l):
    def body(a, wf, bf, wl, bl, h1a_ref, of_ref, ol_ref):
        h1 = a[0] + a[1]
        h1a = jnp.where(h1 >= 0.0, h1, 0.01 * h1)
        h1a_ref[...] = h1a
        of_ref[...] = jnp.dot(h1a, wf[...],
                              preferred_element_type=jnp.float32) + bf[...]
        ol_ref[...] = jnp.dot(h1a, wl[...],
                              preferred_element_type=jnp.float32) + bl[...]

    row = lambda i: (i, 0)
    full = lambda i: (0, 0)
    return pl.pallas_call(
        body,
        grid=(_GRID,),
        in_specs=[
            pl.BlockSpec((2, _BLK, H), lambda i: (0, i, 0)),
            pl.BlockSpec((H, H), full),
            pl.BlockSpec((1, H), full),
            pl.BlockSpec((H, H), full),
            pl.BlockSpec((1, H), full),
        ],
        out_specs=[pl.BlockSpec((_BLK, H), row)] * 3,
        out_shape=[jax.ShapeDtypeStruct((N, H), jnp.float32)] * 3,
    )(acc, W2f, b2f, W2l, b2l)


def _final(acc):
    def body(a, out_ref):
        out_ref[...] = a[0][:, :NCLS] + a[1][:, :NCLS]

    return pl.pallas_call(
        body,
        grid=(_GRID,),
        in_specs=[pl.BlockSpec((2, _BLK, H), lambda i: (0, i, 0))],
        out_specs=pl.BlockSpec((_BLK, NCLS), lambda i: (i, 0)),
        out_shape=jax.ShapeDtypeStruct((N, NCLS), jnp.float32),
    )(acc)


@functools.lru_cache(maxsize=None)
def _make_agg():
    mesh = plsc.VectorSubcoreMesh(core_axis_name="c", subcore_axis_name="s")

    out_type = jax.ShapeDtypeStruct((2, N, H), jnp.float32)

    scratch = [
        pltpu.VMEM((1, CHUNK), jnp.int32),
        pltpu.VMEM((1, CHUNK), jnp.int32),
        pltpu.VMEM((PT, CHUNK), jnp.int32),
        pltpu.VMEM((PT, CHUNK), jnp.int32),
        pltpu.VMEM((2, K, CHUNK, H), jnp.float32),
        pltpu.VMEM((CHUNK,), jnp.float32),
        pltpu.VMEM((ROWS_PER_TILE, H), jnp.float32),
        pltpu.VMEM((ROWS_PER_TILE + 8, ), jnp.float32),
        pltpu.VMEM_SHARED((N, H), jnp.float32),
        pltpu.VMEM_SHARED((N,), jnp.float32),
        pltpu.SemaphoreType.DMA((2,)),
        pltpu.SemaphoreType.DMA((2,)),
        pltpu.SemaphoreType.DMA,
    ]

    def body(tblf, tbll, eif, eil, acc_out, *refs):
        (srcb, dstb, src_all, dst_all, ring, ones, zbuf, zbuf1,
         acc_sp, cnt_sp, sem_g, sem_s, sem_c) = refs

        c = lax.axis_index("c")
        s = lax.axis_index("s")

        zero16 = jnp.zeros((LANES,), jnp.float32)
        start = jnp.where(s == NS - 1, N - ROWS_PER_TILE,
                          s * ROWS_PER_TILE)

        def zrow(i, carry):
            zbuf[i, :] = zero16
            return carry
        lax.fori_loop(0, ROWS_PER_TILE, zrow, 0)
        pltpu.sync_copy(zbuf, acc_sp.at[pl.ds(start, ROWS_PER_TILE)])
        for i in range((ROWS_PER_TILE + 8) // LANES):
            zbuf1[pl.ds(i * LANES, LANES)] = zero16
        pltpu.sync_copy(zbuf1.at[pl.ds(0, ROWS_PER_TILE)],
                        cnt_sp.at[pl.ds(start, ROWS_PER_TILE)])
        one16 = jnp.ones((LANES,), jnp.float32)
        for i in range(CHUNK // LANES):
            ones[pl.ds(i * LANES, LANES)] = one16

        plsc.subcore_barrier()

        def process(tbl, ei):
            pltpu.sync_copy(ei.at[0, pl.ds(s * PT, PT)], src_all)
            pltpu.sync_copy(ei.at[1, pl.ds(s * PT, PT)], dst_all)

            def issue_gathers(g):
                h = g % 2
                for k in range(K):
                    pltpu.async_copy(tbl.at[src_all.at[g * K + k]],
                                     ring.at[h, k], sem_g.at[h])

            def wait_gathers(h):
                for k in range(K):
                    pltpu.make_async_copy(tbl.at[src_all.at[k]],
                                          ring.at[0, k], sem_g.at[h]).wait()

            def issue_scatters(g):
                h = g % 2
                for k in range(K):
                    pltpu.async_copy(ring.at[h, k],
                                     acc_sp.at[dst_all.at[g * K + k]],
                                     sem_s.at[h], add=True)

            def wait_scatters(h):
                for k in range(K):
                    pltpu.make_async_copy(ring.at[0, k],
                                          acc_sp.at[dst_all.at[k]],
                                          sem_s.at[h]).wait()

            def issue_cnts(g):
                for k in range(K):
                    pltpu.async_copy(ones,
                                     cnt_sp.at[dst_all.at[g * K + k]],
                                     sem_c, add=True)

            def wait_cnts():
                for k in range(K):
                    pltpu.make_async_copy(ones, cnt_sp.at[dst_all.at[k]],
                                          sem_c).wait()

            issue_gathers(0)

            def grp(g, carry):
                h = g % 2
                o = (g + 1) % 2
                @pl.when(g >= 1)
                def _():
                    wait_scatters(o)
                @pl.when(g + 1 < NG)
                def _():
                    issue_gathers(g + 1)
                wait_gathers(h)
                issue_scatters(g)
                issue_cnts(g)
                @pl.when(g >= 1)
                def _():
                    wait_cnts()
                return carry
            lax.fori_loop(0, NG, grp, 0)

            wait_scatters((NG - 1) % 2)
            wait_cnts()

            @pl.when(s < LEFT)
            def _():
                ci = NS * PT + s
                pltpu.sync_copy(ei.at[0, ci], srcb.at[0])
                pltpu.sync_copy(ei.at[1, ci], dstb.at[0])
                pltpu.async_copy(tbl.at[srcb.at[0]], ring.at[0, 0],
                                 sem_g.at[0]).wait()
                pltpu.sync_copy(ring.at[0, 0], acc_sp.at[dstb.at[0]],
                                add=True)
                pltpu.sync_copy(ones, cnt_sp.at[dstb.at[0]], add=True)

        @pl.when(c == 0)
        def _():
            process(tblf, eif)

        @pl.when(c == 1)
        def _():
            process(tbll, eil)

        plsc.subcore_barrier()

        pltpu.sync_copy(acc_sp.at[pl.ds(start, ROWS_PER_TILE)], zbuf)
        pltpu.sync_copy(cnt_sp.at[pl.ds(start, ROWS_PER_TILE)],
                        zbuf1.at[pl.ds(0, ROWS_PER_TILE)])

        def divgrp(j, carry):
            rv = 1.0 / jnp.maximum(zbuf1[pl.ds(8 * j, LANES)], 1.0)
            for m in range(8):
                zbuf[8 * j + m, :] = zbuf[8 * j + m, :] * jnp.full(
                    (LANES,), rv[m], jnp.float32)
            return carry
        lax.fori_loop(0, ROWS_PER_TILE // 8, divgrp, 0)

        pltpu.sync_copy(zbuf, acc_out.at[c, pl.ds(start, ROWS_PER_TILE)])

    return pl.kernel(body, out_type=out_type, mesh=mesh,
                     scratch_types=scratch,
                     compiler_params=pltpu.CompilerParams(
                         use_tc_tiling_on_sc=False))


def kernel(x, edge_index_follows, edge_index_likes,
           W1_follows, b1_follows, W1_likes, b1_likes,
           W2_follows, b2_follows, W2_likes, b2_likes):
    eif = edge_index_follows.reshape(2, NCHUNKS, CHUNK)
    eil = edge_index_likes.reshape(2, NCHUNKS, CHUNK)

    W1 = jnp.concatenate([W1_follows, W1_likes], axis=1)
    b1 = jnp.concatenate([b1_follows, b1_likes])[None, :]
    tbl1_f, tbl1_l = _linear1(x, W1, b1)

    agg = _make_agg()
    acc1 = agg(tbl1_f, tbl1_l, eif, eil)

    pad = ((0, 0), (0, H - NCLS))
    W2f = jnp.pad(W2_follows, pad)
    W2l = jnp.pad(W2_likes, pad)
    b2f = jnp.pad(b2_follows, (0, H - NCLS))[None, :]
    b2l = jnp.pad(b2_likes, (0, H - NCLS))[None, :]

    h1_act, tbl2_f, tbl2_l = _mid(acc1, W2f, b2f, W2l, b2l)

    acc2 = agg(tbl2_f, tbl2_l, eif, eil)

    h2 = _final(acc2)
    return (h2, h1_act)


# speedup vs baseline: 39.0793x; 1.0691x over previous
import functools

import jax
import jax.numpy as jnp
from jax import lax
from jax.experimental import pallas as pl
from jax.experimental.pallas import tpu as pltpu
from jax.experimental.pallas import tpu_sc as plsc

N = 10000
E = 320000
DIN = 128
H = 16
NCLS = 2

NC = 2
NS = 16
LANES = 16

CHUNK = 128
NCHUNKS = E // CHUNK
PT = NCHUNKS // NS
LEFT = NCHUNKS - PT * NS
K = 13
NG = PT // K
ROWS_PER_TILE = 632


_BLK = 5000
_GRID = N // _BLK


def _linear1(x, W1, b1):
    def body(x_ref, w_ref, b_ref, of_ref, ol_ref):
        res = jnp.dot(x_ref[...], w_ref[...],
                      preferred_element_type=jnp.float32) + b_ref[...]
        of_ref[...] = res[:, :H]
        ol_ref[...] = res[:, H:]

    return pl.pallas_call(
        body,
        grid=(_GRID,),
        in_specs=[
            pl.BlockSpec((_BLK, DIN), lambda i: (i, 0)),
            pl.BlockSpec((DIN, 2 * H), lambda i: (0, 0)),
            pl.BlockSpec((1, 2 * H), lambda i: (0, 0)),
        ],
        out_specs=[pl.BlockSpec((_BLK, H), lambda i: (i, 0))] * 2,
        out_shape=[jax.ShapeDtypeStruct((N, H), jnp.float32)] * 2,
    )(x, W1, b1)


def _mid(acc, W2f, b2f, W2l, b2l):
    def body(a, wf, bf, wl, bl, h1a_ref, of_ref, ol_ref):
        h1 = a[0] + a[1]
        h1a = jnp.where(h1 >= 0.0, h1, 0.01 * h1)
        h1a_ref[...] = h1a
        of_ref[...] = jnp.dot(h1a, wf[...],
                              preferred_element_type=jnp.float32) + bf[...]
        ol_ref[...] = jnp.dot(h1a, wl[...],
                              preferred_element_type=jnp.float32) + bl[...]

    row = lambda i: (i, 0)
    full = lambda i: (0, 0)
    return pl.pallas_call(
        body,
        grid=(_GRID,),
        in_specs=[
            pl.BlockSpec((2, _BLK, H), lambda i: (0, i, 0)),
            pl.BlockSpec((H, H), full),
            pl.BlockSpec((1, H), full),
            pl.BlockSpec((H, H), full),
            pl.BlockSpec((1, H), full),
        ],
        out_specs=[pl.BlockSpec((_BLK, H), row)] * 3,
        out_shape=[jax.ShapeDtypeStruct((N, H), jnp.float32)] * 3,
    )(acc, W2f, b2f, W2l, b2l)


def _final(acc):
    def body(a, out_ref):
        out_ref[...] = a[0][:, :NCLS] + a[1][:, :NCLS]

    return pl.pallas_call(
        body,
        grid=(_GRID,),
        in_specs=[pl.BlockSpec((2, _BLK, H), lambda i: (0, i, 0))],
        out_specs=pl.BlockSpec((_BLK, NCLS), lambda i: (i, 0)),
        out_shape=jax.ShapeDtypeStruct((N, NCLS), jnp.float32),
    )(acc)


@functools.lru_cache(maxsize=None)
def _make_agg():
    mesh = plsc.VectorSubcoreMesh(core_axis_name="c", subcore_axis_name="s")

    out_type = jax.ShapeDtypeStruct((2, N, H), jnp.float32)

    scratch = [
        pltpu.VMEM((1, CHUNK), jnp.int32),
        pltpu.VMEM((1, CHUNK), jnp.int32),
        pltpu.VMEM((PT, CHUNK), jnp.int32),
        pltpu.VMEM((PT, CHUNK), jnp.int32),
        pltpu.VMEM((2, K, CHUNK, H), jnp.float32),
        pltpu.VMEM((CHUNK,), jnp.float32),
        pltpu.VMEM((ROWS_PER_TILE, H), jnp.float32),
        pltpu.VMEM((ROWS_PER_TILE + 8, ), jnp.float32),
        pltpu.VMEM_SHARED((N, H), jnp.float32),
        pltpu.VMEM_SHARED((N,), jnp.float32),
        pltpu.SemaphoreType.DMA((2,)),
        pltpu.SemaphoreType.DMA((2,)),
        pltpu.SemaphoreType.DMA,
        pltpu.SemaphoreType.DMA,
    ]

    def body(tblf, tbll, eif, eil, acc_out, *refs):
        (srcb, dstb, src_all, dst_all, ring, ones, zbuf, zbuf1,
         acc_sp, cnt_sp, sem_g, sem_s, sem_c, sem_i) = refs

        c = lax.axis_index("c")
        s = lax.axis_index("s")

        @pl.when(c == 0)
        def _():
            pltpu.async_copy(eif.at[0, pl.ds(s * PT, PT)], src_all, sem_i)
            pltpu.async_copy(eif.at[1, pl.ds(s * PT, PT)], dst_all, sem_i)

        @pl.when(c == 1)
        def _():
            pltpu.async_copy(eil.at[0, pl.ds(s * PT, PT)], src_all, sem_i)
            pltpu.async_copy(eil.at[1, pl.ds(s * PT, PT)], dst_all, sem_i)

        zero16 = jnp.zeros((LANES,), jnp.float32)
        start = jnp.where(s == NS - 1, N - ROWS_PER_TILE,
                          s * ROWS_PER_TILE)

        def zrow(i, carry):
            zbuf[i, :] = zero16
            return carry
        lax.fori_loop(0, ROWS_PER_TILE, zrow, 0)
        pltpu.sync_copy(zbuf, acc_sp.at[pl.ds(start, ROWS_PER_TILE)])
        for i in range((ROWS_PER_TILE + 8) // LANES):
            zbuf1[pl.ds(i * LANES, LANES)] = zero16
        pltpu.sync_copy(zbuf1.at[pl.ds(0, ROWS_PER_TILE)],
                        cnt_sp.at[pl.ds(start, ROWS_PER_TILE)])
        one16 = jnp.ones((LANES,), jnp.float32)
        for i in range(CHUNK // LANES):
            ones[pl.ds(i * LANES, LANES)] = one16

        plsc.subcore_barrier()

        def process(tbl, ei):
            pltpu.make_async_copy(ei.at[0, pl.ds(s * PT, PT)], src_all,
                                  sem_i).wait()
            pltpu.make_async_copy(ei.at[1, pl.ds(s * PT, PT)], dst_all,
                                  sem_i).wait()

            def issue_gathers(g):
                h = g % 2
                for k in range(K):
                    pltpu.async_copy(tbl.at[src_all.at[g * K + k]],
                                     ring.at[h, k], sem_g.at[h])

            def wait_gathers(h):
                for k in range(K):
                    pltpu.make_async_copy(tbl.at[src_all.at[k]],
                                          ring.at[0, k], sem_g.at[h]).wait()

            def issue_scatters(g):
                h = g % 2
                for k in range(K):
                    pltpu.async_copy(ring.at[h, k],
                                     acc_sp.at[dst_all.at[g * K + k]],
                                     sem_s.at[h], add=True)

            def wait_scatters(h):
                for k in range(K):
                    pltpu.make_async_copy(ring.at[0, k],
                                          acc_sp.at[dst_all.at[k]],
                                          sem_s.at[h]).wait()

            def issue_cnts(g):
                for k in range(K):
                    pltpu.async_copy(ones,
                                     cnt_sp.at[dst_all.at[g * K + k]],
                                     sem_c, add=True)

            def wait_cnts():
                for k in range(K):
                    pltpu.make_async_copy(ones, cnt_sp.at[dst_all.at[k]],
                                          sem_c).wait()

            issue_gathers(0)

            def grp(g, carry):
                h = g % 2
                o = (g + 1) % 2
                @pl.when(g >= 1)
                def _():
                    wait_scatters(o)
                @pl.when(g + 1 < NG)
                def _():
                    issue_gathers(g + 1)
                wait_gathers(h)
                issue_scatters(g)
                issue_cnts(g)
                @pl.when(g >= 1)
                def _():
                    wait_cnts()
                return carry
            lax.fori_loop(0, NG, grp, 0)

            wait_scatters((NG - 1) % 2)
            wait_cnts()

            @pl.when(s < LEFT)
            def _():
                ci = NS * PT + s
                pltpu.sync_copy(ei.at[0, ci], srcb.at[0])
                pltpu.sync_copy(ei.at[1, ci], dstb.at[0])
                pltpu.async_copy(tbl.at[srcb.at[0]], ring.at[0, 0],
                                 sem_g.at[0]).wait()
                pltpu.sync_copy(ring.at[0, 0], acc_sp.at[dstb.at[0]],
                                add=True)
                pltpu.sync_copy(ones, cnt_sp.at[dstb.at[0]], add=True)

        @pl.when(c == 0)
        def _():
            process(tblf, eif)

        @pl.when(c == 1)
        def _():
            process(tbll, eil)

        plsc.subcore_barrier()

        pltpu.sync_copy(acc_sp.at[pl.ds(start, ROWS_PER_TILE)], zbuf)
        pltpu.sync_copy(cnt_sp.at[pl.ds(start, ROWS_PER_TILE)],
                        zbuf1.at[pl.ds(0, ROWS_PER_TILE)])

        def divgrp(j, carry):
            rv = 1.0 / jnp.maximum(zbuf1[pl.ds(8 * j, LANES)], 1.0)
            for m in range(8):
                zbuf[8 * j + m, :] = zbuf[8 * j + m, :] * jnp.full(
                    (LANES,), rv[m], jnp.float32)
            return carry
        lax.fori_loop(0, ROWS_PER_TILE // 8, divgrp, 0)

        pltpu.sync_copy(zbuf, acc_out.at[c, pl.ds(start, ROWS_PER_TILE)])

    return pl.kernel(body, out_type=out_type, mesh=mesh,
                     scratch_types=scratch,
                     compiler_params=pltpu.CompilerParams(
                         use_tc_tiling_on_sc=False))


def kernel(x, edge_index_follows, edge_index_likes,
           W1_follows, b1_follows, W1_likes, b1_likes,
           W2_follows, b2_follows, W2_likes, b2_likes):
    eif = edge_index_follows.reshape(2, NCHUNKS, CHUNK)
    eil = edge_index_likes.reshape(2, NCHUNKS, CHUNK)

    W1 = jnp.concatenate([W1_follows, W1_likes], axis=1)
    b1 = jnp.concatenate([b1_follows, b1_likes])[None, :]
    tbl1_f, tbl1_l = _linear1(x, W1, b1)

    agg = _make_agg()
    acc1 = agg(tbl1_f, tbl1_l, eif, eil)

    pad = ((0, 0), (0, H - NCLS))
    W2f = jnp.pad(W2_follows, pad)
    W2l = jnp.pad(W2_likes, pad)
    b2f = jnp.pad(b2_follows, (0, H - NCLS))[None, :]
    b2l = jnp.pad(b2_likes, (0, H - NCLS))[None, :]

    h1_act, tbl2_f, tbl2_l = _mid(acc1, W2f, b2f, W2l, b2l)

    acc2 = agg(tbl2_f, tbl2_l, eif, eil)

    h2 = _final(acc2)
    return (h2, h1_act)
